# baseline (speedup 1.0000x reference)
import functools

import jax
import jax.numpy as jnp
from jax import lax
from jax.experimental import pallas as pl
from jax.experimental.pallas import tpu as pltpu
from jax.experimental.pallas import tpu_sc as plsc

B, C, H, W = 8, 192, 224, 224
HW = H * W
ROWS = B * C
C2 = 2 * C
TOPK = 64
NPLANES = B * TOPK

_RB = 32
_HR = ROWS // 2
_GR = _HR // _RB


def _pool_body(a_ref, b_ref, sa_ref, sb_ref):
    sa_ref[0, 0, :] = jnp.sum(a_ref[...], axis=1)
    sb_ref[0, 0, :] = jnp.sum(b_ref[...], axis=1)


_NW = 32
_PRW = ROWS // _NW


_HWH = HW // 2


def _sc_pool_body(xh_hbm, ps_hbm, b0, b1, b2, b3, sums_v, g0, g1, g2, g3):
    cid = lax.axis_index("c")
    sid = lax.axis_index("s")
    wid = sid * 2 + cid
    h0 = wid * _PRW * 2
    nh = _PRW * 2
    bufs = (b0, b1, b2, b3)
    sems = (g0, g1, g2, g3)

    def issue(h, k):
        pltpu.make_async_copy(
            xh_hbm.at[pl.ds(h0 + h, 1)], bufs[k], sems[k]).start()

    def wait(h, k):
        pltpu.make_async_copy(
            xh_hbm.at[pl.ds(h0 + h, 1)], bufs[k], sems[k]).wait()

    def half_acc(buf, accs):
        def inner(i, a4):
            a = list(a4)
            base = i * 256
            for k in range(16):
                a[k % 4] = a[k % 4] + buf[0, pl.ds(base + k * 16, 16)]
            return tuple(a)

        return lax.fori_loop(0, _HWH // 256, inner, accs, unroll=7)

    for k in range(4):
        issue(k, k)

    z = jnp.zeros((16,), jnp.float32)

    def row_pair(i, carry):
        h = 4 * i

        def do_row(rr, k0, k1):
            accs = (z, z, z, z)
            wait(h + k0, k0)
            accs = half_acc(bufs[k0], accs)

            @pl.when(h + k0 + 4 < nh)
            def _():
                issue(h + k0 + 4, k0)

            wait(h + k1, k1)
            accs = half_acc(bufs[k1], accs)

            @pl.when(h + k1 + 4 < nh)
            def _():
                issue(h + k1 + 4, k1)

            a0, a1, a2, a3 = accs
            sums_v[pl.ds(rr * 16, 16)] = (a0 + a1) + (a2 + a3)

        do_row(2 * i, 0, 1)
        do_row(2 * i + 1, 2, 3)
        return carry

    lax.fori_loop(0, _PRW // 2, row_pair, 0)
    pltpu.sync_copy(sums_v,
                    ps_hbm.at[pl.ds(wid * (_PRW * 16), _PRW * 16)])


def _make_sc_pool():
    return functools.partial(
        pl.kernel,
        mesh=plsc.VectorSubcoreMesh(core_axis_name="c", subcore_axis_name="s"),
        out_type=jax.ShapeDtypeStruct((ROWS * 16,), jnp.float32),
        scratch_types=[
            pltpu.VMEM((1, _HWH), jnp.float32),
            pltpu.VMEM((1, _HWH), jnp.float32),
            pltpu.VMEM((1, _HWH), jnp.float32),
            pltpu.VMEM((1, _HWH), jnp.float32),
            pltpu.VMEM((_PRW * 16,), jnp.float32),
            pltpu.SemaphoreType.DMA,
            pltpu.SemaphoreType.DMA,
            pltpu.SemaphoreType.DMA,
            pltpu.SemaphoreType.DMA,
        ],
    )(_sc_pool_body)


def _topk_body(s0_ref, p1_ref, r0_ref, r1_ref, u0_ref):
    fold = (lax.broadcasted_iota(jnp.int32, (C * 16, C), 0) // 16 ==
            lax.broadcasted_iota(jnp.int32, (C * 16, C), 1)
            ).astype(jnp.float32)
    s1 = jnp.dot(p1_ref[...], fold, preferred_element_type=jnp.float32,
                 precision=lax.Precision.HIGHEST)
    vals = jnp.concatenate([s0_ref[...], s1], axis=1) / float(HW)
    iota_c = lax.broadcasted_iota(jnp.int32, (B, C2), 1)
    iota_k = lax.broadcasted_iota(jnp.int32, (B, TOPK), 1)
    idxm = jnp.zeros((B, TOPK), jnp.int32)
    for k in range(TOPK):
        m = jnp.max(vals, axis=1, keepdims=True)
        cand = jnp.where(vals == m, iota_c, jnp.int32(2**30))
        sel = jnp.min(cand, axis=1)
        idxm = jnp.where(iota_k == k, sel[:, None], idxm)
        vals = jnp.where(iota_c == sel[:, None], -jnp.inf, vals)
    rowbase = lax.broadcasted_iota(jnp.int32, (B, TOPK), 0) * C
    r0_ref[...] = rowbase + jnp.minimum(idxm, C - 1)
    r1_ref[...] = rowbase + jnp.maximum(idxm - C, 0)
    u0_ref[...] = (idxm < C).astype(jnp.int32)


_NW = 32
_PPW = NPLANES // _NW


def _sc_gather_body(r0_hbm, r1_hbm, u0_hbm, x0_hbm, x1_hbm, o_hbm,
                    idx_v, b0, b1, g0, g1, s0, s1):
    cid = lax.axis_index("c")
    sid = lax.axis_index("s")
    wid = sid * 2 + cid
    base = wid * _PPW
    pltpu.sync_copy(r0_hbm.at[pl.ds(base, _PPW)], idx_v.at[0])
    pltpu.sync_copy(r1_hbm.at[pl.ds(base, _PPW)], idx_v.at[1])
    pltpu.sync_copy(u0_hbm.at[pl.ds(base, _PPW)], idx_v.at[2])
    bufs = (b0, b1)
    gsems = (g0, g1)
    ssems = (s0, s1)

    r0v = idx_v[0]
    r1v = idx_v[1]
    u0v = idx_v[2]
    rv = jnp.where(u0v == 1, r0v, r1v)

    def row(j):
        return rv[j]

    def issue_gather(j):
        r = row(j)
        u = u0v[j]
        buf, sem = bufs[j % 2], gsems[j % 2]

        @pl.when(u == 1)
        def _():
            pltpu.make_async_copy(x0_hbm.at[pl.ds(r, 1)], buf, sem).start()

        @pl.when(u == 0)
        def _():
            pltpu.make_async_copy(x1_hbm.at[pl.ds(r, 1)], buf, sem).start()

    def wait_gather(j):
        pltpu.make_async_copy(x0_hbm.at[pl.ds(row(j), 1)], bufs[j % 2],
                              gsems[j % 2]).wait()

    def issue_scatter(j):
        pltpu.make_async_copy(bufs[j % 2], o_hbm.at[pl.ds(base + j, 1)],
                              ssems[j % 2]).start()

    def wait_scatter(j):
        pltpu.make_async_copy(bufs[j % 2], o_hbm.at[pl.ds(base + j, 1)],
                              ssems[j % 2]).wait()

    issue_gather(0)
    issue_gather(1)
    for j in range(_PPW):
        wait_gather(j)
        issue_scatter(j)
        if j + 2 < _PPW:
            wait_scatter(j)
            issue_gather(j + 2)
    wait_scatter(_PPW - 2)
    wait_scatter(_PPW - 1)


def kernel(x_0, x_1):
    x0r = x_0.reshape(ROWS, HW)
    x1r = x_1.reshape(ROWS, HW)

    ps = _make_sc_pool()(x_1.reshape(ROWS * 2, _HWH))

    sa, sb = pl.pallas_call(
        _pool_body,
        grid=(_GR,),
        in_specs=[pl.BlockSpec((_RB, HW), lambda i: (i, 0)),
                  pl.BlockSpec((_RB, HW), lambda i: (i + _GR, 0))],
        out_specs=[pl.BlockSpec((1, 1, _RB), lambda i: (i, 0, 0))] * 2,
        out_shape=[jax.ShapeDtypeStruct((_GR, 1, _RB), jnp.float32)] * 2,
        compiler_params=pltpu.CompilerParams(
            dimension_semantics=("arbitrary",)),
    )(x0r, x0r)
    s0 = jnp.concatenate([sa.reshape(-1), sb.reshape(-1)])

    r0, r1, u0 = pl.pallas_call(
        _topk_body,
        out_shape=[jax.ShapeDtypeStruct((B, TOPK), jnp.int32)] * 3,
    )(s0.reshape(B, C), ps.reshape(B, C * 16))

    sc_gather = functools.partial(
        pl.kernel,
        mesh=plsc.VectorSubcoreMesh(core_axis_name="c", subcore_axis_name="s"),
        out_type=jax.ShapeDtypeStruct((NPLANES, HW), jnp.float32),
        scratch_types=[
            pltpu.VMEM((3, _PPW), jnp.int32),
            pltpu.VMEM((1, HW), jnp.float32),
            pltpu.VMEM((1, HW), jnp.float32),
            pltpu.SemaphoreType.DMA,
            pltpu.SemaphoreType.DMA,
            pltpu.SemaphoreType.DMA,
            pltpu.SemaphoreType.DMA,
        ],
    )(_sc_gather_body)
    out = sc_gather(r0.reshape(-1), r1.reshape(-1), u0.reshape(-1), x0r, x1r)

    return out.reshape(B, TOPK, H, W)


# speedup vs baseline: 1.3337x; 1.3337x over previous
import functools

import jax
import jax.numpy as jnp
from jax import lax
from jax.experimental import pallas as pl
from jax.experimental.pallas import tpu as pltpu
from jax.experimental.pallas import tpu_sc as plsc

B, C, H, W = 8, 192, 224, 224
HW = H * W
ROWS = B * C
C2 = 2 * C
TOPK = 64
NPLANES = B * TOPK

_RB = 64
_HR = ROWS // 2
_GR = _HR // _RB


def _pool_body(a_ref, b_ref, sa_ref, sb_ref):
    sa_ref[0, 0, :] = jnp.sum(a_ref[...], axis=1)
    sb_ref[0, 0, :] = jnp.sum(b_ref[...], axis=1)


_NW = 32
_PRW = ROWS // _NW


def _sc_pool_body(xr_hbm, ps_hbm, b0, b1, sums_v, g0, g1):
    cid = lax.axis_index("c")
    sid = lax.axis_index("s")
    wid = sid * 2 + cid
    row0 = wid * _PRW

    def issue(r, buf, sem):
        pltpu.make_async_copy(
            xr_hbm.at[pl.ds(row0 + r, 1)], buf, sem).start()

    def wait(r, buf, sem):
        pltpu.make_async_copy(
            xr_hbm.at[pl.ds(row0 + r, 1)], buf, sem).wait()

    def reduce_row(buf, r):
        z = jnp.zeros((16,), jnp.float32)

        def inner(i, accs):
            a = list(accs)
            base = i * 256
            for k in range(16):
                a[k % 4] = a[k % 4] + buf[0, pl.ds(base + k * 16, 16)]
            return tuple(a)

        a0, a1, a2, a3 = lax.fori_loop(0, HW // 256, inner, (z, z, z, z),
                                       unroll=7)
        sums_v[pl.ds(r * 16, 16)] = (a0 + a1) + (a2 + a3)

    issue(0, b0, g0)

    def outer(i, carry):
        r0 = 2 * i
        r1 = r0 + 1
        wait(r0, b0, g0)
        issue(r1, b1, g1)
        reduce_row(b0, r0)
        wait(r1, b1, g1)

        @pl.when(r1 + 1 < _PRW)
        def _():
            issue(r1 + 1, b0, g0)

        reduce_row(b1, r1)
        return carry

    lax.fori_loop(0, _PRW // 2, outer, 0)
    pltpu.sync_copy(sums_v,
                    ps_hbm.at[pl.ds(wid * (_PRW * 16), _PRW * 16)])


def _make_sc_pool():
    return functools.partial(
        pl.kernel,
        mesh=plsc.VectorSubcoreMesh(core_axis_name="c", subcore_axis_name="s"),
        out_type=jax.ShapeDtypeStruct((ROWS * 16,), jnp.float32),
        scratch_types=[
            pltpu.VMEM((1, HW), jnp.float32),
            pltpu.VMEM((1, HW), jnp.float32),
            pltpu.VMEM((_PRW * 16,), jnp.float32),
            pltpu.SemaphoreType.DMA,
            pltpu.SemaphoreType.DMA,
        ],
    )(_sc_pool_body)


def _topk_body(s0_ref, p1_ref, r0_ref, r1_ref, u0_ref):
    fold = (lax.broadcasted_iota(jnp.int32, (C * 16, C), 0) // 16 ==
            lax.broadcasted_iota(jnp.int32, (C * 16, C), 1)
            ).astype(jnp.float32)
    s1 = jnp.dot(p1_ref[...], fold, preferred_element_type=jnp.float32,
                 precision=lax.Precision.HIGHEST)
    vals = jnp.concatenate([s0_ref[...], s1], axis=1) / float(HW)
    iota_c = lax.broadcasted_iota(jnp.int32, (B, C2), 1)
    iota_k = lax.broadcasted_iota(jnp.int32, (B, TOPK), 1)
    idxm = jnp.zeros((B, TOPK), jnp.int32)
    for k in range(TOPK):
        m = jnp.max(vals, axis=1, keepdims=True)
        cand = jnp.where(vals == m, iota_c, jnp.int32(2**30))
        sel = jnp.min(cand, axis=1)
        idxm = jnp.where(iota_k == k, sel[:, None], idxm)
        vals = jnp.where(iota_c == sel[:, None], -jnp.inf, vals)
    rowbase = lax.broadcasted_iota(jnp.int32, (B, TOPK), 0) * C
    r0_ref[...] = rowbase + jnp.minimum(idxm, C - 1)
    r1_ref[...] = rowbase + jnp.maximum(idxm - C, 0)
    u0_ref[...] = (idxm < C).astype(jnp.int32)


_NW = 32
_PPW = NPLANES // _NW


def _sc_gather_body(r0_hbm, r1_hbm, u0_hbm, x0_hbm, x1_hbm, o_hbm,
                    idx_v, b0, b1, g0, g1, s0, s1):
    cid = lax.axis_index("c")
    sid = lax.axis_index("s")
    wid = sid * 2 + cid
    base = wid * _PPW
    pltpu.sync_copy(r0_hbm.at[pl.ds(base, _PPW)], idx_v.at[0])
    pltpu.sync_copy(r1_hbm.at[pl.ds(base, _PPW)], idx_v.at[1])
    pltpu.sync_copy(u0_hbm.at[pl.ds(base, _PPW)], idx_v.at[2])
    bufs = (b0, b1)
    gsems = (g0, g1)
    ssems = (s0, s1)

    r0v = idx_v[0]
    r1v = idx_v[1]
    u0v = idx_v[2]
    rv = jnp.where(u0v == 1, r0v, r1v)

    def row(j):
        return rv[j]

    def issue_gather(j):
        r = row(j)
        u = u0v[j]
        buf, sem = bufs[j % 2], gsems[j % 2]

        @pl.when(u == 1)
        def _():
            pltpu.make_async_copy(x0_hbm.at[pl.ds(r, 1)], buf, sem).start()

        @pl.when(u == 0)
        def _():
            pltpu.make_async_copy(x1_hbm.at[pl.ds(r, 1)], buf, sem).start()

    def wait_gather(j):
        pltpu.make_async_copy(x0_hbm.at[pl.ds(row(j), 1)], bufs[j % 2],
                              gsems[j % 2]).wait()

    def issue_scatter(j):
        pltpu.make_async_copy(bufs[j % 2], o_hbm.at[pl.ds(base + j, 1)],
                              ssems[j % 2]).start()

    def wait_scatter(j):
        pltpu.make_async_copy(bufs[j % 2], o_hbm.at[pl.ds(base + j, 1)],
                              ssems[j % 2]).wait()

    issue_gather(0)
    issue_gather(1)
    for j in range(_PPW):
        wait_gather(j)
        issue_scatter(j)
        if j + 2 < _PPW:
            wait_scatter(j)
            issue_gather(j + 2)
    wait_scatter(_PPW - 2)
    wait_scatter(_PPW - 1)


def kernel(x_0, x_1):
    x0r = x_0.reshape(ROWS, HW)
    x1r = x_1.reshape(ROWS, HW)

    ps = _make_sc_pool()(x1r)

    sa, sb = pl.pallas_call(
        _pool_body,
        grid=(_GR,),
        in_specs=[pl.BlockSpec((_RB, HW), lambda i: (i, 0)),
                  pl.BlockSpec((_RB, HW), lambda i: (i + _GR, 0))],
        out_specs=[pl.BlockSpec((1, 1, _RB), lambda i: (i, 0, 0))] * 2,
        out_shape=[jax.ShapeDtypeStruct((_GR, 1, _RB), jnp.float32)] * 2,
        compiler_params=pltpu.CompilerParams(
            dimension_semantics=("arbitrary",)),
    )(x0r, x0r)
    s0 = jnp.concatenate([sa.reshape(-1), sb.reshape(-1)])

    r0, r1, u0 = pl.pallas_call(
        _topk_body,
        out_shape=[jax.ShapeDtypeStruct((B, TOPK), jnp.int32)] * 3,
    )(s0.reshape(B, C), ps.reshape(B, C * 16))

    sc_gather = functools.partial(
        pl.kernel,
        mesh=plsc.VectorSubcoreMesh(core_axis_name="c", subcore_axis_name="s"),
        out_type=jax.ShapeDtypeStruct((NPLANES, HW), jnp.float32),
        scratch_types=[
            pltpu.VMEM((3, _PPW), jnp.int32),
            pltpu.VMEM((1, HW), jnp.float32),
            pltpu.VMEM((1, HW), jnp.float32),
            pltpu.SemaphoreType.DMA,
            pltpu.SemaphoreType.DMA,
            pltpu.SemaphoreType.DMA,
            pltpu.SemaphoreType.DMA,
        ],
    )(_sc_gather_body)
    out = sc_gather(r0.reshape(-1), r1.reshape(-1), u0.reshape(-1), x0r, x1r)

    return out.reshape(B, TOPK, H, W)


# speedup vs baseline: 1.3348x; 1.0008x over previous
import functools

import jax
import jax.numpy as jnp
from jax import lax
from jax.experimental import pallas as pl
from jax.experimental.pallas import tpu as pltpu
from jax.experimental.pallas import tpu_sc as plsc

B, C, H, W = 8, 192, 224, 224
HW = H * W
ROWS = B * C
C2 = 2 * C
TOPK = 64
NPLANES = B * TOPK

_TRB = 16
_TNC = ROWS // _TRB


def _pool_body(x_any, s_ref, b0, b1, b2, b3, t0, t1, t2, t3):
    bufs = (b0, b1, b2, b3)
    sems = (t0, t1, t2, t3)

    def issue(c, k):
        pltpu.make_async_copy(x_any.at[pl.ds(c * _TRB, _TRB), :],
                              bufs[k], sems[k]).start()

    def wait(c, k):
        pltpu.make_async_copy(x_any.at[pl.ds(c * _TRB, _TRB), :],
                              bufs[k], sems[k]).wait()

    for k in range(4):
        issue(k, k)

    def outer(i, carry):
        c = 4 * i
        for k in range(4):
            wait(c + k, k)
            s_ref[pl.ds(c + k, 1), :] = jnp.sum(bufs[k][...], axis=1)[None, :]

            @pl.when(c + k + 4 < _TNC)
            def _():
                issue(c + k + 4, k)
        return carry

    lax.fori_loop(0, _TNC // 4, outer, 0)


_NW = 32
_PRW = ROWS // _NW


def _sc_pool_body(xr_hbm, ps_hbm, b0, b1, sums_v, g0, g1):
    cid = lax.axis_index("c")
    sid = lax.axis_index("s")
    wid = sid * 2 + cid
    row0 = wid * _PRW

    def issue(r, buf, sem):
        pltpu.make_async_copy(
            xr_hbm.at[pl.ds(row0 + r, 1)], buf, sem).start()

    def wait(r, buf, sem):
        pltpu.make_async_copy(
            xr_hbm.at[pl.ds(row0 + r, 1)], buf, sem).wait()

    def reduce_row(buf, r):
        z = jnp.zeros((16,), jnp.float32)

        def inner(i, accs):
            a = list(accs)
            base = i * 256
            for k in range(16):
                a[k % 4] = a[k % 4] + buf[0, pl.ds(base + k * 16, 16)]
            return tuple(a)

        a0, a1, a2, a3 = lax.fori_loop(0, HW // 256, inner, (z, z, z, z),
                                       unroll=7)
        sums_v[pl.ds(r * 16, 16)] = (a0 + a1) + (a2 + a3)

    issue(0, b0, g0)

    def outer(i, carry):
        r0 = 2 * i
        r1 = r0 + 1
        wait(r0, b0, g0)
        issue(r1, b1, g1)
        reduce_row(b0, r0)
        wait(r1, b1, g1)

        @pl.when(r1 + 1 < _PRW)
        def _():
            issue(r1 + 1, b0, g0)

        reduce_row(b1, r1)
        return carry

    lax.fori_loop(0, _PRW // 2, outer, 0)
    pltpu.sync_copy(sums_v,
                    ps_hbm.at[pl.ds(wid * (_PRW * 16), _PRW * 16)])


def _make_sc_pool():
    return functools.partial(
        pl.kernel,
        mesh=plsc.VectorSubcoreMesh(core_axis_name="c", subcore_axis_name="s"),
        out_type=jax.ShapeDtypeStruct((ROWS * 16,), jnp.float32),
        scratch_types=[
            pltpu.VMEM((1, HW), jnp.float32),
            pltpu.VMEM((1, HW), jnp.float32),
            pltpu.VMEM((_PRW * 16,), jnp.float32),
            pltpu.SemaphoreType.DMA,
            pltpu.SemaphoreType.DMA,
        ],
    )(_sc_pool_body)


def _topk_body(s0_ref, p1_ref, r0_ref, r1_ref, u0_ref):
    fold = (lax.broadcasted_iota(jnp.int32, (C * 16, C), 0) // 16 ==
            lax.broadcasted_iota(jnp.int32, (C * 16, C), 1)
            ).astype(jnp.float32)
    s1 = jnp.dot(p1_ref[...], fold, preferred_element_type=jnp.float32,
                 precision=lax.Precision.HIGHEST)
    vals = jnp.concatenate([s0_ref[...], s1], axis=1) / float(HW)
    iota_c = lax.broadcasted_iota(jnp.int32, (B, C2), 1)
    iota_k = lax.broadcasted_iota(jnp.int32, (B, TOPK), 1)
    idxm = jnp.zeros((B, TOPK), jnp.int32)
    for k in range(TOPK):
        m = jnp.max(vals, axis=1, keepdims=True)
        cand = jnp.where(vals == m, iota_c, jnp.int32(2**30))
        sel = jnp.min(cand, axis=1)
        idxm = jnp.where(iota_k == k, sel[:, None], idxm)
        vals = jnp.where(iota_c == sel[:, None], -jnp.inf, vals)
    rowbase = lax.broadcasted_iota(jnp.int32, (B, TOPK), 0) * C
    r0_ref[...] = rowbase + jnp.minimum(idxm, C - 1)
    r1_ref[...] = rowbase + jnp.maximum(idxm - C, 0)
    u0_ref[...] = (idxm < C).astype(jnp.int32)


_NW = 32
_PPW = NPLANES // _NW


def _sc_gather_body(r0_hbm, r1_hbm, u0_hbm, x0_hbm, x1_hbm, o_hbm,
                    idx_v, b0, b1, g0, g1, s0, s1):
    cid = lax.axis_index("c")
    sid = lax.axis_index("s")
    wid = sid * 2 + cid
    base = wid * _PPW
    pltpu.sync_copy(r0_hbm.at[pl.ds(base, _PPW)], idx_v.at[0])
    pltpu.sync_copy(r1_hbm.at[pl.ds(base, _PPW)], idx_v.at[1])
    pltpu.sync_copy(u0_hbm.at[pl.ds(base, _PPW)], idx_v.at[2])
    bufs = (b0, b1)
    gsems = (g0, g1)
    ssems = (s0, s1)

    r0v = idx_v[0]
    r1v = idx_v[1]
    u0v = idx_v[2]
    rv = jnp.where(u0v == 1, r0v, r1v)

    def row(j):
        return rv[j]

    def issue_gather(j):
        r = row(j)
        u = u0v[j]
        buf, sem = bufs[j % 2], gsems[j % 2]

        @pl.when(u == 1)
        def _():
            pltpu.make_async_copy(x0_hbm.at[pl.ds(r, 1)], buf, sem).start()

        @pl.when(u == 0)
        def _():
            pltpu.make_async_copy(x1_hbm.at[pl.ds(r, 1)], buf, sem).start()

    def wait_gather(j):
        pltpu.make_async_copy(x0_hbm.at[pl.ds(row(j), 1)], bufs[j % 2],
                              gsems[j % 2]).wait()

    def issue_scatter(j):
        pltpu.make_async_copy(bufs[j % 2], o_hbm.at[pl.ds(base + j, 1)],
                              ssems[j % 2]).start()

    def wait_scatter(j):
        pltpu.make_async_copy(bufs[j % 2], o_hbm.at[pl.ds(base + j, 1)],
                              ssems[j % 2]).wait()

    issue_gather(0)
    issue_gather(1)
    for j in range(_PPW):
        wait_gather(j)
        issue_scatter(j)
        if j + 2 < _PPW:
            wait_scatter(j)
            issue_gather(j + 2)
    wait_scatter(_PPW - 2)
    wait_scatter(_PPW - 1)


def kernel(x_0, x_1):
    x0r = x_0.reshape(ROWS, HW)
    x1r = x_1.reshape(ROWS, HW)

    ps = _make_sc_pool()(x1r)

    s0 = pl.pallas_call(
        _pool_body,
        in_specs=[pl.BlockSpec(memory_space=pl.ANY)],
        out_specs=pl.BlockSpec(memory_space=pltpu.VMEM),
        out_shape=jax.ShapeDtypeStruct((_TNC, _TRB), jnp.float32),
        scratch_shapes=[pltpu.VMEM((_TRB, HW), jnp.float32)] * 4
        + [pltpu.SemaphoreType.DMA] * 4,
    )(x0r)

    r0, r1, u0 = pl.pallas_call(
        _topk_body,
        out_shape=[jax.ShapeDtypeStruct((B, TOPK), jnp.int32)] * 3,
    )(s0.reshape(B, C), ps.reshape(B, C * 16))

    sc_gather = functools.partial(
        pl.kernel,
        mesh=plsc.VectorSubcoreMesh(core_axis_name="c", subcore_axis_name="s"),
        out_type=jax.ShapeDtypeStruct((NPLANES, HW), jnp.float32),
        scratch_types=[
            pltpu.VMEM((3, _PPW), jnp.int32),
            pltpu.VMEM((1, HW), jnp.float32),
            pltpu.VMEM((1, HW), jnp.float32),
            pltpu.SemaphoreType.DMA,
            pltpu.SemaphoreType.DMA,
            pltpu.SemaphoreType.DMA,
            pltpu.SemaphoreType.DMA,
        ],
    )(_sc_gather_body)
    out = sc_gather(r0.reshape(-1), r1.reshape(-1), u0.reshape(-1), x0r, x1r)

    return out.reshape(B, TOPK, H, W)
